# baseline (speedup 1.0000x reference)
import functools

import jax
import jax.numpy as jnp
from jax import lax
from jax.experimental import pallas as pl
from jax.experimental.pallas import tpu as pltpu
from jax.experimental.pallas import tpu_sc as plsc

N = 10000
E = 320000
D_IN = 128
FH1 = 256
FH2 = 128
GH1 = 128
GH2 = 64
NC = 10
LAT = GH2 + FH2

_F32 = jnp.float32


def _elu(h):
    return jnp.where(h > 0, h, jnp.exp(h) - 1.0)


def _bn(h, g, b):
    m = jnp.mean(h, axis=0)
    v = jnp.mean((h - m) ** 2, axis=0)
    return g * (h - m) * jax.lax.rsqrt(v + 1e-4) + b


def _encoder_body(x_ref, w1_ref, b1_ref, g1_ref, be1_ref,
                  w2_ref, b2_ref, g2_ref, be2_ref,
                  wl_ref, wr_ref, out_ref, xl_ref, xr_ref):
    x = x_ref[...]
    h = jnp.dot(x, w1_ref[...], preferred_element_type=_F32) + b1_ref[...]
    h = _elu(_bn(h, g1_ref[...], be1_ref[...]))
    h = jnp.dot(h, w2_ref[...], preferred_element_type=_F32) + b2_ref[...]
    feat = _elu(_bn(h, g2_ref[...], be2_ref[...]))
    out_ref[...] = feat
    xl_ref[...] = jnp.dot(feat, wl_ref[...], preferred_element_type=_F32)
    xr_ref[...] = jnp.dot(feat, wr_ref[...], preferred_element_type=_F32)


def _encoder(x, W1, b1, g1, be1, W2, b2, g2, be2, Wl1, Wr1):
    return pl.pallas_call(
        _encoder_body,
        out_shape=[
            jax.ShapeDtypeStruct((N, FH2), _F32),
            jax.ShapeDtypeStruct((N, GH1), _F32),
            jax.ShapeDtypeStruct((N, GH1), _F32),
        ],
    )(x, W1, b1, g1, be1, W2, b2, g2, be2, Wl1, Wr1)


def _combine_proj_body(num_ref, den_ref, bias_ref, wl_ref, wr_ref,
                       xl_ref, xr_ref):
    num = num_ref[0] + num_ref[1]
    den = den_ref[0] + den_ref[1]
    h = num / (den[:, None] + 1e-16) + bias_ref[...]
    h = _elu(h)
    xl_ref[...] = jnp.dot(h, wl_ref[...], preferred_element_type=_F32)
    xr_ref[...] = jnp.dot(h, wr_ref[...], preferred_element_type=_F32)


def _combine_proj(num, den, bias, Wl, Wr, gh_out):
    return pl.pallas_call(
        _combine_proj_body,
        out_shape=[
            jax.ShapeDtypeStruct((N, gh_out), _F32),
            jax.ShapeDtypeStruct((N, gh_out), _F32),
        ],
    )(num, den, bias, Wl, Wr)


def _combine_proj4_body(num_ref, den_ref, bias_ref,
                        wl2_ref, wr2_ref, wl3_ref, wr3_ref,
                        xl2_ref, xr2_ref, xl3_ref, xr3_ref):
    num = num_ref[0] + num_ref[1]
    den = den_ref[0] + den_ref[1]
    h = num / (den[:, None] + 1e-16) + bias_ref[...]
    h = _elu(h)
    xl2_ref[...] = jnp.dot(h, wl2_ref[...], preferred_element_type=_F32)
    xr2_ref[...] = jnp.dot(h, wr2_ref[...], preferred_element_type=_F32)
    xl3_ref[...] = jnp.dot(h, wl3_ref[...], preferred_element_type=_F32)
    xr3_ref[...] = jnp.dot(h, wr3_ref[...], preferred_element_type=_F32)


def _combine_proj4(num, den, bias, Wl2, Wr2, Wl3, Wr3):
    return pl.pallas_call(
        _combine_proj4_body,
        out_shape=[jax.ShapeDtypeStruct((N, GH2), _F32)] * 4,
    )(num, den, bias, Wl2, Wr2, Wl3, Wr3)


def _combine2_body(num2_ref, den2_ref, bi2_ref, num3_ref, den3_ref, bi3_ref,
                   mu_ref, lv_ref):
    mu_ref[...] = (num2_ref[0] + num2_ref[1]) / (
        (den2_ref[0] + den2_ref[1])[:, None] + 1e-16) + bi2_ref[...]
    lv_ref[...] = (num3_ref[0] + num3_ref[1]) / (
        (den3_ref[0] + den3_ref[1])[:, None] + 1e-16) + bi3_ref[...]


def _combine2(num2, den2, bi2, num3, den3, bi3):
    return pl.pallas_call(
        _combine2_body,
        out_shape=[jax.ShapeDtypeStruct((N, GH2), _F32)] * 2,
    )(num2, den2, bi2, num3, den3, bi3)


def _decoder_body(z_ref, wd_ref, bd_ref, gd_ref, bed_ref, cl_ref,
                  de_ref, q_ref):
    z = z_ref[...]
    h = jnp.dot(z, wd_ref[...], preferred_element_type=_F32) + bd_ref[...]
    de_ref[...] = _elu(_bn(h, gd_ref[...], bed_ref[...]))
    cl = cl_ref[...]
    d2 = (jnp.sum(z * z, axis=1, keepdims=True)
          - 2.0 * jnp.dot(z, cl.T, preferred_element_type=_F32)
          + jnp.sum(cl * cl, axis=1)[None, :])
    q = 1.0 / (1.0 + d2)
    q_ref[...] = q / jnp.sum(q, axis=1, keepdims=True)


def _decoder(z, Wd, bd, gd, bed, cluster):
    return pl.pallas_call(
        _decoder_body,
        out_shape=[
            jax.ShapeDtypeStruct((N, D_IN), _F32),
            jax.ShapeDtypeStruct((N, NC), _F32),
        ],
    )(z, Wd, bd, gd, bed, cluster)


NP = 10240
_NW = 32
_RPT = NP // 16

_CFG = {128: (16, 632), 64: (128, 80)}


def _make_sc_edge(w, heads):
    _G, _TPW = _CFG[w]
    _GB = 128 // _G
    _TB = _TPW // _GB
    hw = w // heads
    mesh = plsc.VectorSubcoreMesh(core_axis_name="c", subcore_axis_name="s")

    @functools.partial(
        pl.kernel,
        mesh=mesh,
        compiler_params=pltpu.CompilerParams(
            needs_layout_passes=False,
            use_tc_tiling_on_sc=True),
        out_type=([jax.ShapeDtypeStruct((2, NP, w), _F32)]
                  + [jax.ShapeDtypeStruct((2, NP), _F32)] * heads),
        scratch_types=(
            [pltpu.VMEM((_TB, 2, 128), jnp.int32)]
            + [pltpu.VMEM((_G, w), _F32)] * 6
            + [pltpu.VMEM((_G,), _F32)] * (2 * heads)
            + [pltpu.VMEM((_G,), jnp.int32)] * 2
            + [pltpu.VMEM((w,), _F32)]
            + [pltpu.VMEM_SHARED((NP, w), _F32)]
            + [pltpu.VMEM_SHARED((NP,), _F32)] * heads
            + [pltpu.SemaphoreType.DMA] * 4),
    )
    def edge_kernel(xl_hbm, xr_hbm, sdw_hbm, att_hbm, zrow_hbm, zden_hbm,
                    *rest):
        num_hbm = rest[0]
        den_hbms = rest[1:1 + heads]
        sc = rest[1 + heads:]
        idx_all, rl0, rl1, rr0, rr1, ob0, ob1 = sc[:7]
        PBH = [(sc[7 + 2 * h], sc[8 + 2 * h]) for h in range(heads)]
        is0, is1, att_v, num_sh = sc[7 + 2 * heads:11 + 2 * heads]
        den_shs = sc[11 + 2 * heads:11 + 3 * heads]
        sg0, sg1, ss0, ss1 = sc[11 + 3 * heads:]

        cid = lax.axis_index("c")
        sid = lax.axis_index("s")
        wid = sid * 2 + cid
        r0 = sid * _RPT
        RL, RR, OB = (rl0, rl1), (rr0, rr1), (ob0, ob1)
        IS, SG, SS = (is0, is1), (sg0, sg1), (ss0, ss1)

        pltpu.sync_copy(zrow_hbm, num_sh.at[pl.ds(r0, _RPT)])
        for h in range(heads):
            pltpu.sync_copy(zden_hbm, den_shs[h].at[pl.ds(r0, _RPT)])
        pltpu.sync_copy(att_hbm, att_v)
        pltpu.sync_copy(sdw_hbm.at[wid], idx_all)
        plsc.subcore_barrier()

        lanes = lax.iota(jnp.int32, 16)

        def gather_idx(t, which):
            tb = t // _GB
            off = (t % _GB) * _G
            return idx_all.at[tb, which, pl.ds(off, _G)]

        def fire_gathers(b, t):
            pltpu.make_async_copy(
                xl_hbm.at[gather_idx(t, 0)], RL[b], SG[b]).start()
            pltpu.make_async_copy(
                xr_hbm.at[gather_idx(t, 1)], RR[b], SG[b]).start()

        def wait_gathers(b, t):
            pltpu.make_async_copy(
                xl_hbm.at[gather_idx(t, 0)], RL[b], SG[b]).wait()
            pltpu.make_async_copy(
                xr_hbm.at[gather_idx(t, 1)], RR[b], SG[b]).wait()

        def fire_scatters(b):
            pltpu.make_async_copy(
                OB[b], num_sh.at[IS[b]], SS[b]).start(add=True)
            for h in range(heads):
                pltpu.make_async_copy(
                    PBH[h][b], den_shs[h].at[IS[b]], SS[b]).start(add=True)

        def wait_scatters(b):
            pltpu.make_async_copy(OB[b], num_sh.at[IS[b]], SS[b]).wait()
            for h in range(heads):
                pltpu.make_async_copy(
                    PBH[h][b], den_shs[h].at[IS[b]], SS[b]).wait()

        def copy_scatter_idx(b, t):
            tb = t // _GB
            off = (t % _GB) * _G
            for k in range(_G // 16):
                IS[b][pl.ds(k * 16, 16)] = idx_all[
                    tb, 1, pl.ds(off + k * 16, 16)]

        def compute(b):
            rl, rr, ob = RL[b], RR[b], OB[b]
            att_regs = [att_v[pl.ds(k * 16, 16)] for k in range(w // 16)]
            kph = hw // 16

            def batch_body(bb, carry2):
                e0 = bb * 16
                pvs = [jnp.zeros((16,), _F32) for _ in range(heads)]
                for i in range(16):
                    for h in range(heads):
                        acc = None
                        for k in range(h * kph, (h + 1) * kph):
                            vl = rl[e0 + i, pl.ds(k * 16, 16)]
                            vr = rr[e0 + i, pl.ds(k * 16, 16)]
                            u = vl + vr
                            u = jnp.maximum(u, 0.2 * u)
                            t = u * att_regs[k]
                            acc = t if acc is None else acc + t
                        pvs[h] = jnp.where(lanes == i, jnp.sum(acc), pvs[h])
                pvs = [jnp.exp(pv) for pv in pvs]
                for h in range(heads):
                    PBH[h][b][pl.ds(e0, 16)] = pvs[h]
                for i in range(16):
                    for h in range(heads):
                        p = pvs[h][i]
                        for k in range(h * kph, (h + 1) * kph):
                            ob[e0 + i, pl.ds(k * 16, 16)] = (
                                rl[e0 + i, pl.ds(k * 16, 16)] * p)
                return carry2

            lax.fori_loop(0, _G // 16, batch_body, 0)

        for b in range(2):
            fire_gathers(b, b)

        def outer_body(t2, carry):
            for b in range(2):
                t = 2 * t2 + b
                wait_gathers(b, t)

                @pl.when(t2 > 0)
                def _():
                    wait_scatters(b)

                copy_scatter_idx(b, t)
                compute(b)
                fire_scatters(b)

                @pl.when(t2 < (_TPW // 2) - 1)
                def _():
                    fire_gathers(b, t + 2)
            return carry

        lax.fori_loop(0, _TPW // 2, outer_body, 0)
        for b in range(2):
            wait_scatters(b)

        plsc.subcore_barrier()
        pltpu.sync_copy(num_sh.at[pl.ds(r0, _RPT)],
                        num_hbm.at[cid, pl.ds(r0, _RPT)])
        for h in range(heads):
            pltpu.sync_copy(den_shs[h].at[pl.ds(r0, _RPT)],
                            den_hbms[h].at[cid, pl.ds(r0, _RPT)])

    return edge_kernel


@functools.lru_cache(maxsize=None)
def _get_sc_edge(w, heads):
    return _make_sc_edge(w, heads)


def _pack_edges(src, dst, w):
    g, tpw = _CFG[w]
    gb = 128 // g
    ep = _NW * tpw * g
    pad = jnp.full((ep - E,), N, jnp.int32)
    src_p = jnp.concatenate([src, pad])
    dst_p = jnp.concatenate([dst, pad])
    sd = jnp.stack([src_p.reshape(ep // g, g), dst_p.reshape(ep // g, g)],
                   axis=1)
    sd = sd.reshape(tpw, _NW, 2, g).transpose(1, 0, 2, 3)
    return sd.reshape(_NW, tpw // gb, gb, 2, g).transpose(
        0, 1, 3, 2, 4).reshape(_NW, tpw // gb, 2, 128)


def _edge(xl, xr, sdw, att, heads=1):
    w = int(xl.shape[1])
    fn = _get_sc_edge(w, heads)
    xl_p = jnp.pad(xl, ((0, NP - N), (0, 0)))
    xr_p = jnp.pad(xr, ((0, NP - N), (0, 0)))
    zrow = jnp.zeros((_RPT, w), _F32)
    zden = jnp.zeros((_RPT,), _F32)
    outs = fn(xl_p, xr_p, sdw, att, zrow, zden)
    num = outs[0][:, :N]
    dens = [d[:, :N] for d in outs[1:]]
    return (num, *dens)


def kernel(x, adj, x_hr, training, W1, b1, g1, be1, W2, b2, g2, be2,
           Wl1, Wr1, a1, bi1, Wlh, Wrh, ah, bih,
           Wl2, Wr2, a2, bi2, Wl3, Wr3, a3, bi3,
           Wd, bd, gd, bed, cluster):
    sdw128 = _pack_edges(adj[0], adj[1], GH1)

    feat_x, xl1, xr1 = _encoder(x, W1, b1, g1, be1, W2, b2, g2, be2, Wl1, Wr1)

    num1, den1 = _edge(xl1, xr1, sdw128, a1)
    xlh, xrh = _combine_proj(num1, den1, bi1, Wlh, Wrh, GH1)

    numh, denh = _edge(xlh, xrh, sdw128, ah)
    Wlc = jnp.concatenate([Wl2, Wl3], axis=1)
    Wrc = jnp.concatenate([Wr2, Wr3], axis=1)
    attc = jnp.concatenate([a2, a3])
    xlc, xrc = _combine_proj(numh, denh, bih, Wlc, Wrc, 2 * GH2)
    num23, den2, den3 = _edge(xlc, xrc, sdw128, attc, heads=2)
    num2 = num23[:, :, :GH2]
    num3 = num23[:, :, GH2:]

    mu, logvar = _combine2(num2, den2, bi2, num3, den3, bi3)
    z = jnp.concatenate([feat_x, mu], axis=1)
    de_feat, q = _decoder(z, Wd, bd, gd, bed, cluster)
    return (z, mu, logvar, de_feat, q, feat_x, mu)


# speedup vs baseline: 1.1491x; 1.1491x over previous
import functools

import jax
import jax.numpy as jnp
from jax import lax
from jax.experimental import pallas as pl
from jax.experimental.pallas import tpu as pltpu
from jax.experimental.pallas import tpu_sc as plsc

N = 10000
E = 320000
D_IN = 128
FH1 = 256
FH2 = 128
GH1 = 128
GH2 = 64
NC = 10
LAT = GH2 + FH2

_F32 = jnp.float32


def _elu(h):
    return jnp.where(h > 0, h, jnp.exp(h) - 1.0)


def _bn(h, g, b):
    m = jnp.mean(h, axis=0)
    v = jnp.mean((h - m) ** 2, axis=0)
    return g * (h - m) * jax.lax.rsqrt(v + 1e-4) + b


def _encoder_body(x_ref, w1_ref, b1_ref, g1_ref, be1_ref,
                  w2_ref, b2_ref, g2_ref, be2_ref,
                  wl_ref, wr_ref, out_ref, xl_ref, xr_ref):
    x = x_ref[...]
    h = jnp.dot(x, w1_ref[...], preferred_element_type=_F32) + b1_ref[...]
    h = _elu(_bn(h, g1_ref[...], be1_ref[...]))
    h = jnp.dot(h, w2_ref[...], preferred_element_type=_F32) + b2_ref[...]
    feat = _elu(_bn(h, g2_ref[...], be2_ref[...]))
    out_ref[...] = feat
    xl_ref[...] = jnp.dot(feat, wl_ref[...], preferred_element_type=_F32)
    xr_ref[...] = jnp.dot(feat, wr_ref[...], preferred_element_type=_F32)


def _encoder(x, W1, b1, g1, be1, W2, b2, g2, be2, Wl1, Wr1):
    return pl.pallas_call(
        _encoder_body,
        out_shape=[
            jax.ShapeDtypeStruct((N, FH2), _F32),
            jax.ShapeDtypeStruct((N, GH1), _F32),
            jax.ShapeDtypeStruct((N, GH1), _F32),
        ],
    )(x, W1, b1, g1, be1, W2, b2, g2, be2, Wl1, Wr1)


def _combine_proj_body(num_ref, den_ref, bias_ref, wl_ref, wr_ref,
                       xl_ref, xr_ref):
    num = num_ref[0] + num_ref[1]
    den = den_ref[0] + den_ref[1]
    h = num / (den[:, None] + 1e-16) + bias_ref[...]
    h = _elu(h)
    xl_ref[...] = jnp.dot(h, wl_ref[...], preferred_element_type=_F32)
    xr_ref[...] = jnp.dot(h, wr_ref[...], preferred_element_type=_F32)


def _combine_proj(num, den, bias, Wl, Wr, gh_out):
    return pl.pallas_call(
        _combine_proj_body,
        out_shape=[
            jax.ShapeDtypeStruct((N, gh_out), _F32),
            jax.ShapeDtypeStruct((N, gh_out), _F32),
        ],
    )(num, den, bias, Wl, Wr)


def _combine2_body(num2_ref, den2_ref, bi2_ref, num3_ref, den3_ref, bi3_ref,
                   mu_ref, lv_ref):
    mu_ref[...] = (num2_ref[0] + num2_ref[1]) / (
        (den2_ref[0] + den2_ref[1])[:, None] + 1e-16) + bi2_ref[...]
    lv_ref[...] = (num3_ref[0] + num3_ref[1]) / (
        (den3_ref[0] + den3_ref[1])[:, None] + 1e-16) + bi3_ref[...]


def _combine2(num2, den2, bi2, num3, den3, bi3):
    return pl.pallas_call(
        _combine2_body,
        out_shape=[jax.ShapeDtypeStruct((N, GH2), _F32)] * 2,
    )(num2, den2, bi2, num3, den3, bi3)


def _decoder_body(z_ref, wd_ref, bd_ref, gd_ref, bed_ref, cl_ref,
                  de_ref, q_ref):
    z = z_ref[...]
    h = jnp.dot(z, wd_ref[...], preferred_element_type=_F32) + bd_ref[...]
    de_ref[...] = _elu(_bn(h, gd_ref[...], bed_ref[...]))
    cl = cl_ref[...]
    d2 = (jnp.sum(z * z, axis=1, keepdims=True)
          - 2.0 * jnp.dot(z, cl.T, preferred_element_type=_F32)
          + jnp.sum(cl * cl, axis=1)[None, :])
    q = 1.0 / (1.0 + d2)
    q_ref[...] = q / jnp.sum(q, axis=1, keepdims=True)


def _decoder(z, Wd, bd, gd, bed, cluster):
    return pl.pallas_call(
        _decoder_body,
        out_shape=[
            jax.ShapeDtypeStruct((N, D_IN), _F32),
            jax.ShapeDtypeStruct((N, NC), _F32),
        ],
    )(z, Wd, bd, gd, bed, cluster)


NP = 10240
_NW = 32
_RPT = NP // 16

_CFG = {128: (32, 316)}


def _make_sc_edge(w, heads):
    _G, _TPW = _CFG[w]
    _GB = 128 // _G
    _TB = _TPW // _GB
    hw = w // heads
    mesh = plsc.VectorSubcoreMesh(core_axis_name="c", subcore_axis_name="s")

    @functools.partial(
        pl.kernel,
        mesh=mesh,
        compiler_params=pltpu.CompilerParams(
            needs_layout_passes=False,
            use_tc_tiling_on_sc=True),
        out_type=([jax.ShapeDtypeStruct((2, NP, w), _F32)]
                  + [jax.ShapeDtypeStruct((2, NP), _F32)] * heads),
        scratch_types=(
            [pltpu.VMEM((_TB, 2, 128), jnp.int32)]
            + [pltpu.VMEM((_G, w), _F32)] * 6
            + [pltpu.VMEM((_G,), _F32)] * (2 * heads)
            + [pltpu.VMEM((_G,), jnp.int32)] * 2
            + [pltpu.VMEM((w,), _F32)]
            + [pltpu.VMEM_SHARED((NP, w), _F32)]
            + [pltpu.VMEM_SHARED((NP,), _F32)] * heads
            + [pltpu.SemaphoreType.DMA] * 4),
    )
    def edge_kernel(xl_hbm, xr_hbm, sdw_hbm, att_hbm, zrow_hbm, zden_hbm,
                    *rest):
        num_hbm = rest[0]
        den_hbms = rest[1:1 + heads]
        sc = rest[1 + heads:]
        idx_all, rl0, rl1, rr0, rr1, ob0, ob1 = sc[:7]
        PBH = [(sc[7 + 2 * h], sc[8 + 2 * h]) for h in range(heads)]
        is0, is1, att_v, num_sh = sc[7 + 2 * heads:11 + 2 * heads]
        den_shs = sc[11 + 2 * heads:11 + 3 * heads]
        sg0, sg1, ss0, ss1 = sc[11 + 3 * heads:]

        cid = lax.axis_index("c")
        sid = lax.axis_index("s")
        wid = sid * 2 + cid
        r0 = sid * _RPT
        RL, RR, OB = (rl0, rl1), (rr0, rr1), (ob0, ob1)
        IS, SG, SS = (is0, is1), (sg0, sg1), (ss0, ss1)

        pltpu.sync_copy(zrow_hbm, num_sh.at[pl.ds(r0, _RPT)])
        for h in range(heads):
            pltpu.sync_copy(zden_hbm, den_shs[h].at[pl.ds(r0, _RPT)])
        pltpu.sync_copy(att_hbm, att_v)
        pltpu.sync_copy(sdw_hbm.at[wid], idx_all)
        plsc.subcore_barrier()

        lanes = lax.iota(jnp.int32, 16)

        def gather_idx(t, which):
            tb = t // _GB
            off = (t % _GB) * _G
            return idx_all.at[tb, which, pl.ds(off, _G)]

        def fire_gathers(b, t):
            pltpu.make_async_copy(
                xl_hbm.at[gather_idx(t, 0)], RL[b], SG[b]).start()
            pltpu.make_async_copy(
                xr_hbm.at[gather_idx(t, 1)], RR[b], SG[b]).start()

        def wait_gathers(b, t):
            pltpu.make_async_copy(
                xl_hbm.at[gather_idx(t, 0)], RL[b], SG[b]).wait()
            pltpu.make_async_copy(
                xr_hbm.at[gather_idx(t, 1)], RR[b], SG[b]).wait()

        def fire_scatters(b):
            pltpu.make_async_copy(
                OB[b], num_sh.at[IS[b]], SS[b]).start(add=True)
            for h in range(heads):
                pltpu.make_async_copy(
                    PBH[h][b], den_shs[h].at[IS[b]], SS[b]).start(add=True)

        def wait_scatters(b):
            pltpu.make_async_copy(OB[b], num_sh.at[IS[b]], SS[b]).wait()
            for h in range(heads):
                pltpu.make_async_copy(
                    PBH[h][b], den_shs[h].at[IS[b]], SS[b]).wait()

        def copy_scatter_idx(b, t):
            tb = t // _GB
            off = (t % _GB) * _G
            for k in range(_G // 16):
                IS[b][pl.ds(k * 16, 16)] = idx_all[
                    tb, 1, pl.ds(off + k * 16, 16)]

        def compute(b):
            rl, rr, ob = RL[b], RR[b], OB[b]
            att_regs = [att_v[pl.ds(k * 16, 16)] for k in range(w // 16)]
            kph = hw // 16

            def batch_body(bb, carry2):
                e0 = bb * 16
                pvs = [jnp.zeros((16,), _F32) for _ in range(heads)]
                for i in range(16):
                    for h in range(heads):
                        acc = None
                        for k in range(h * kph, (h + 1) * kph):
                            vl = rl[e0 + i, pl.ds(k * 16, 16)]
                            vr = rr[e0 + i, pl.ds(k * 16, 16)]
                            u = vl + vr
                            u = jnp.maximum(u, 0.2 * u)
                            t = u * att_regs[k]
                            acc = t if acc is None else acc + t
                        pvs[h] = jnp.where(lanes == i, jnp.sum(acc), pvs[h])
                pvs = [jnp.exp(pv) for pv in pvs]
                for h in range(heads):
                    PBH[h][b][pl.ds(e0, 16)] = pvs[h]
                for i in range(16):
                    for h in range(heads):
                        p = pvs[h][i]
                        for k in range(h * kph, (h + 1) * kph):
                            ob[e0 + i, pl.ds(k * 16, 16)] = (
                                rl[e0 + i, pl.ds(k * 16, 16)] * p)
                return carry2

            lax.fori_loop(0, _G // 16, batch_body, 0)

        for b in range(2):
            fire_gathers(b, b)

        def outer_body(t2, carry):
            for b in range(2):
                t = 2 * t2 + b
                wait_gathers(b, t)

                @pl.when(t2 > 0)
                def _():
                    wait_scatters(b)

                copy_scatter_idx(b, t)
                compute(b)
                fire_scatters(b)

                @pl.when(t2 < (_TPW // 2) - 1)
                def _():
                    fire_gathers(b, t + 2)
            return carry

        lax.fori_loop(0, _TPW // 2, outer_body, 0)
        for b in range(2):
            wait_scatters(b)

        plsc.subcore_barrier()
        pltpu.sync_copy(num_sh.at[pl.ds(r0, _RPT)],
                        num_hbm.at[cid, pl.ds(r0, _RPT)])
        for h in range(heads):
            pltpu.sync_copy(den_shs[h].at[pl.ds(r0, _RPT)],
                            den_hbms[h].at[cid, pl.ds(r0, _RPT)])

    return edge_kernel


@functools.lru_cache(maxsize=None)
def _get_sc_edge(w, heads):
    return _make_sc_edge(w, heads)


def _pack_edges(src, dst, w):
    g, tpw = _CFG[w]
    gb = 128 // g
    ep = _NW * tpw * g
    pad = jnp.full((ep - E,), N, jnp.int32)
    src_p = jnp.concatenate([src, pad])
    dst_p = jnp.concatenate([dst, pad])
    sd = jnp.stack([src_p.reshape(ep // g, g), dst_p.reshape(ep // g, g)],
                   axis=1)
    sd = sd.reshape(tpw, _NW, 2, g).transpose(1, 0, 2, 3)
    return sd.reshape(_NW, tpw // gb, gb, 2, g).transpose(
        0, 1, 3, 2, 4).reshape(_NW, tpw // gb, 2, 128)


def _edge(xl, xr, sdw, att, heads=1):
    w = int(xl.shape[1])
    fn = _get_sc_edge(w, heads)
    xl_p = jnp.pad(xl, ((0, NP - N), (0, 0)))
    xr_p = jnp.pad(xr, ((0, NP - N), (0, 0)))
    zrow = jnp.zeros((_RPT, w), _F32)
    zden = jnp.zeros((_RPT,), _F32)
    outs = fn(xl_p, xr_p, sdw, att, zrow, zden)
    num = outs[0][:, :N]
    dens = [d[:, :N] for d in outs[1:]]
    return (num, *dens)


def kernel(x, adj, x_hr, training, W1, b1, g1, be1, W2, b2, g2, be2,
           Wl1, Wr1, a1, bi1, Wlh, Wrh, ah, bih,
           Wl2, Wr2, a2, bi2, Wl3, Wr3, a3, bi3,
           Wd, bd, gd, bed, cluster):
    sdw128 = _pack_edges(adj[0], adj[1], GH1)

    feat_x, xl1, xr1 = _encoder(x, W1, b1, g1, be1, W2, b2, g2, be2, Wl1, Wr1)

    num1, den1 = _edge(xl1, xr1, sdw128, a1)
    xlh, xrh = _combine_proj(num1, den1, bi1, Wlh, Wrh, GH1)

    numh, denh = _edge(xlh, xrh, sdw128, ah)
    Wlc = jnp.concatenate([Wl2, Wl3], axis=1)
    Wrc = jnp.concatenate([Wr2, Wr3], axis=1)
    attc = jnp.concatenate([a2, a3])
    xlc, xrc = _combine_proj(numh, denh, bih, Wlc, Wrc, 2 * GH2)
    num23, den2, den3 = _edge(xlc, xrc, sdw128, attc, heads=2)
    num2 = num23[:, :, :GH2]
    num3 = num23[:, :, GH2:]

    mu, logvar = _combine2(num2, den2, bi2, num3, den3, bi3)
    z = jnp.concatenate([feat_x, mu], axis=1)
    de_feat, q = _decoder(z, Wd, bd, gd, bed, cluster)
    return (z, mu, logvar, de_feat, q, feat_x, mu)


# speedup vs baseline: 1.2079x; 1.0512x over previous
import functools

import jax
import jax.numpy as jnp
from jax import lax
from jax.experimental import pallas as pl
from jax.experimental.pallas import tpu as pltpu
from jax.experimental.pallas import tpu_sc as plsc

N = 10000
E = 320000
D_IN = 128
FH1 = 256
FH2 = 128
GH1 = 128
GH2 = 64
NC = 10
LAT = GH2 + FH2

_F32 = jnp.float32


def _elu(h):
    return jnp.where(h > 0, h, jnp.exp(h) - 1.0)


def _bn(h, g, b):
    m = jnp.mean(h, axis=0)
    v = jnp.mean((h - m) ** 2, axis=0)
    return g * (h - m) * jax.lax.rsqrt(v + 1e-4) + b


def _encoder_body(x_ref, w1_ref, b1_ref, g1_ref, be1_ref,
                  w2_ref, b2_ref, g2_ref, be2_ref,
                  wl_ref, wr_ref, out_ref, xl_ref, xr_ref):
    x = x_ref[...]
    h = jnp.dot(x, w1_ref[...], preferred_element_type=_F32) + b1_ref[...]
    h = _elu(_bn(h, g1_ref[...], be1_ref[...]))
    h = jnp.dot(h, w2_ref[...], preferred_element_type=_F32) + b2_ref[...]
    feat = _elu(_bn(h, g2_ref[...], be2_ref[...]))
    out_ref[...] = feat
    xl_ref[...] = jnp.dot(feat, wl_ref[...], preferred_element_type=_F32)
    xr_ref[...] = jnp.dot(feat, wr_ref[...], preferred_element_type=_F32)


def _encoder(x, W1, b1, g1, be1, W2, b2, g2, be2, Wl1, Wr1):
    return pl.pallas_call(
        _encoder_body,
        out_shape=[
            jax.ShapeDtypeStruct((N, FH2), _F32),
            jax.ShapeDtypeStruct((N, GH1), _F32),
            jax.ShapeDtypeStruct((N, GH1), _F32),
        ],
    )(x, W1, b1, g1, be1, W2, b2, g2, be2, Wl1, Wr1)


def _combine_proj_body(num_ref, den_ref, bias_ref, wl_ref, wr_ref,
                       xl_ref, xr_ref):
    num = num_ref[0] + num_ref[1]
    den = den_ref[0] + den_ref[1]
    h = num / (den[:, None] + 1e-16) + bias_ref[...]
    h = _elu(h)
    xl_ref[...] = jnp.dot(h, wl_ref[...], preferred_element_type=_F32)
    xr_ref[...] = jnp.dot(h, wr_ref[...], preferred_element_type=_F32)


def _combine_proj(num, den, bias, Wl, Wr, gh_out):
    return pl.pallas_call(
        _combine_proj_body,
        out_shape=[
            jax.ShapeDtypeStruct((N, gh_out), _F32),
            jax.ShapeDtypeStruct((N, gh_out), _F32),
        ],
    )(num, den, bias, Wl, Wr)


def _combine2_body(num2_ref, den2_ref, bi2_ref, num3_ref, den3_ref, bi3_ref,
                   mu_ref, lv_ref):
    mu_ref[...] = (num2_ref[0] + num2_ref[1]) / (
        (den2_ref[0] + den2_ref[1])[:, None] + 1e-16) + bi2_ref[...]
    lv_ref[...] = (num3_ref[0] + num3_ref[1]) / (
        (den3_ref[0] + den3_ref[1])[:, None] + 1e-16) + bi3_ref[...]


def _combine2(num2, den2, bi2, num3, den3, bi3):
    return pl.pallas_call(
        _combine2_body,
        out_shape=[jax.ShapeDtypeStruct((N, GH2), _F32)] * 2,
    )(num2, den2, bi2, num3, den3, bi3)


def _decoder_body(z_ref, wd_ref, bd_ref, gd_ref, bed_ref, cl_ref,
                  de_ref, q_ref):
    z = z_ref[...]
    h = jnp.dot(z, wd_ref[...], preferred_element_type=_F32) + bd_ref[...]
    de_ref[...] = _elu(_bn(h, gd_ref[...], bed_ref[...]))
    cl = cl_ref[...]
    d2 = (jnp.sum(z * z, axis=1, keepdims=True)
          - 2.0 * jnp.dot(z, cl.T, preferred_element_type=_F32)
          + jnp.sum(cl * cl, axis=1)[None, :])
    q = 1.0 / (1.0 + d2)
    q_ref[...] = q / jnp.sum(q, axis=1, keepdims=True)


def _decoder(z, Wd, bd, gd, bed, cluster):
    return pl.pallas_call(
        _decoder_body,
        out_shape=[
            jax.ShapeDtypeStruct((N, D_IN), _F32),
            jax.ShapeDtypeStruct((N, NC), _F32),
        ],
    )(z, Wd, bd, gd, bed, cluster)


NP = 10240
_NW = 32
_RPT = NP // 16

_CFG = {128: (32, 316)}


def _make_sc_edge(w, heads):
    _G, _TPW = _CFG[w]
    _GB = 128 // _G
    _TB = _TPW // _GB
    hw = w // heads
    mesh = plsc.VectorSubcoreMesh(core_axis_name="c", subcore_axis_name="s")

    @functools.partial(
        pl.kernel,
        mesh=mesh,
        compiler_params=pltpu.CompilerParams(
            needs_layout_passes=False,
            use_tc_tiling_on_sc=True),
        out_type=([jax.ShapeDtypeStruct((2, NP, w), _F32)]
                  + [jax.ShapeDtypeStruct((2, NP), _F32)] * heads),
        scratch_types=(
            [pltpu.VMEM((_TB, 2, 128), jnp.int32)]
            + [pltpu.VMEM((_G, w), _F32)] * 6
            + [pltpu.VMEM((_G,), _F32)] * (2 * heads)
            + [pltpu.VMEM((_G,), jnp.int32)] * 2
            + [pltpu.VMEM((w,), _F32)]
            + [pltpu.VMEM_SHARED((NP, w), _F32)]
            + [pltpu.VMEM_SHARED((NP,), _F32)] * heads
            + [pltpu.SemaphoreType.DMA] * 4),
    )
    def edge_kernel(xl_hbm, xr_hbm, sdw_hbm, att_hbm, zrow_hbm, zden_hbm,
                    *rest):
        num_hbm = rest[0]
        den_hbms = rest[1:1 + heads]
        sc = rest[1 + heads:]
        idx_all, rl0, rl1, rr0, rr1, ob0, ob1 = sc[:7]
        PBH = [(sc[7 + 2 * h], sc[8 + 2 * h]) for h in range(heads)]
        is0, is1, att_v, num_sh = sc[7 + 2 * heads:11 + 2 * heads]
        den_shs = sc[11 + 2 * heads:11 + 3 * heads]
        sg0, sg1, ss0, ss1 = sc[11 + 3 * heads:]

        cid = lax.axis_index("c")
        sid = lax.axis_index("s")
        wid = sid * 2 + cid
        r0 = sid * _RPT
        RL, RR, OB = (rl0, rl1), (rr0, rr1), (ob0, ob1)
        IS, SG, SS = (is0, is1), (sg0, sg1), (ss0, ss1)

        pltpu.sync_copy(zrow_hbm, num_sh.at[pl.ds(r0, _RPT)])
        for h in range(heads):
            pltpu.sync_copy(zden_hbm, den_shs[h].at[pl.ds(r0, _RPT)])
        pltpu.sync_copy(att_hbm, att_v)
        pltpu.sync_copy(sdw_hbm.at[wid], idx_all)
        plsc.subcore_barrier()

        lanes = lax.iota(jnp.int32, 16)

        def gather_idx(t, which):
            tb = t // _GB
            off = (t % _GB) * _G
            return idx_all.at[tb, which, pl.ds(off, _G)]

        def fire_gather_l(b, t):
            pltpu.make_async_copy(
                xl_hbm.at[gather_idx(t, 0)], RL[b], SG[b]).start()

        def fire_gather_r(b, t):
            pltpu.make_async_copy(
                xr_hbm.at[gather_idx(t, 1)], RR[b], SG[b]).start()

        def wait_gathers(b, t):
            pltpu.make_async_copy(
                xl_hbm.at[gather_idx(t, 0)], RL[b], SG[b]).wait()
            pltpu.make_async_copy(
                xr_hbm.at[gather_idx(t, 1)], RR[b], SG[b]).wait()

        def fire_scatters(b):
            pltpu.make_async_copy(
                OB[b], num_sh.at[IS[b]], SS[b]).start(add=True)
            for h in range(heads):
                pltpu.make_async_copy(
                    PBH[h][b], den_shs[h].at[IS[b]], SS[b]).start(add=True)

        def wait_scatters(b):
            pltpu.make_async_copy(OB[b], num_sh.at[IS[b]], SS[b]).wait()
            for h in range(heads):
                pltpu.make_async_copy(
                    PBH[h][b], den_shs[h].at[IS[b]], SS[b]).wait()

        def copy_scatter_idx(b, t):
            tb = t // _GB
            off = (t % _GB) * _G
            for k in range(_G // 16):
                IS[b][pl.ds(k * 16, 16)] = idx_all[
                    tb, 1, pl.ds(off + k * 16, 16)]

        def compute_dot(b):
            rl, rr = RL[b], RR[b]
            att_regs = [att_v[pl.ds(k * 16, 16)] for k in range(w // 16)]
            kph = hw // 16

            def batch_body(bb, carry2):
                e0 = bb * 16
                pvs = [jnp.zeros((16,), _F32) for _ in range(heads)]
                for i in range(16):
                    for h in range(heads):
                        acc = None
                        for k in range(h * kph, (h + 1) * kph):
                            vl = rl[e0 + i, pl.ds(k * 16, 16)]
                            vr = rr[e0 + i, pl.ds(k * 16, 16)]
                            u = vl + vr
                            u = jnp.maximum(u, 0.2 * u)
                            t = u * att_regs[k]
                            acc = t if acc is None else acc + t
                        pvs[h] = jnp.where(lanes == i, jnp.sum(acc), pvs[h])
                for h in range(heads):
                    PBH[h][b][pl.ds(e0, 16)] = jnp.exp(pvs[h])
                return carry2

            lax.fori_loop(0, _G // 16, batch_body, 0)

        def compute_scale(b):
            rl, ob = RL[b], OB[b]
            kph = hw // 16

            def batch_body(bb, carry2):
                e0 = bb * 16
                pvs = [PBH[h][b][pl.ds(e0, 16)] for h in range(heads)]
                for i in range(16):
                    for h in range(heads):
                        p = pvs[h][i]
                        for k in range(h * kph, (h + 1) * kph):
                            ob[e0 + i, pl.ds(k * 16, 16)] = (
                                rl[e0 + i, pl.ds(k * 16, 16)] * p)
                return carry2

            lax.fori_loop(0, _G // 16, batch_body, 0)

        for b in range(2):
            fire_gather_l(b, b)
            fire_gather_r(b, b)

        def outer_body(t2, carry):
            for b in range(2):
                t = 2 * t2 + b
                wait_gathers(b, t)

                @pl.when(t2 > 0)
                def _():
                    wait_scatters(b)

                copy_scatter_idx(b, t)
                compute_dot(b)

                @pl.when(t2 < (_TPW // 2) - 1)
                def _():
                    fire_gather_r(b, t + 2)

                compute_scale(b)

                @pl.when(t2 < (_TPW // 2) - 1)
                def _():
                    fire_gather_l(b, t + 2)

                fire_scatters(b)
            return carry

        lax.fori_loop(0, _TPW // 2, outer_body, 0)
        for b in range(2):
            wait_scatters(b)

        plsc.subcore_barrier()
        pltpu.sync_copy(num_sh.at[pl.ds(r0, _RPT)],
                        num_hbm.at[cid, pl.ds(r0, _RPT)])
        for h in range(heads):
            pltpu.sync_copy(den_shs[h].at[pl.ds(r0, _RPT)],
                            den_hbms[h].at[cid, pl.ds(r0, _RPT)])

    return edge_kernel


@functools.lru_cache(maxsize=None)
def _get_sc_edge(w, heads):
    return _make_sc_edge(w, heads)


def _pack_edges(src, dst, w):
    g, tpw = _CFG[w]
    gb = 128 // g
    ep = _NW * tpw * g
    pad = jnp.full((ep - E,), N, jnp.int32)
    src_p = jnp.concatenate([src, pad])
    dst_p = jnp.concatenate([dst, pad])
    sd = jnp.stack([src_p.reshape(ep // g, g), dst_p.reshape(ep // g, g)],
                   axis=1)
    sd = sd.reshape(tpw, _NW, 2, g).transpose(1, 0, 2, 3)
    return sd.reshape(_NW, tpw // gb, gb, 2, g).transpose(
        0, 1, 3, 2, 4).reshape(_NW, tpw // gb, 2, 128)


def _edge(xl, xr, sdw, att, heads=1):
    w = int(xl.shape[1])
    fn = _get_sc_edge(w, heads)
    xl_p = jnp.pad(xl, ((0, NP - N), (0, 0)))
    xr_p = jnp.pad(xr, ((0, NP - N), (0, 0)))
    zrow = jnp.zeros((_RPT, w), _F32)
    zden = jnp.zeros((_RPT,), _F32)
    outs = fn(xl_p, xr_p, sdw, att, zrow, zden)
    num = outs[0][:, :N]
    dens = [d[:, :N] for d in outs[1:]]
    return (num, *dens)


def kernel(x, adj, x_hr, training, W1, b1, g1, be1, W2, b2, g2, be2,
           Wl1, Wr1, a1, bi1, Wlh, Wrh, ah, bih,
           Wl2, Wr2, a2, bi2, Wl3, Wr3, a3, bi3,
           Wd, bd, gd, bed, cluster):
    sdw128 = _pack_edges(adj[0], adj[1], GH1)

    feat_x, xl1, xr1 = _encoder(x, W1, b1, g1, be1, W2, b2, g2, be2, Wl1, Wr1)

    num1, den1 = _edge(xl1, xr1, sdw128, a1)
    xlh, xrh = _combine_proj(num1, den1, bi1, Wlh, Wrh, GH1)

    numh, denh = _edge(xlh, xrh, sdw128, ah)
    Wlc = jnp.concatenate([Wl2, Wl3], axis=1)
    Wrc = jnp.concatenate([Wr2, Wr3], axis=1)
    attc = jnp.concatenate([a2, a3])
    xlc, xrc = _combine_proj(numh, denh, bih, Wlc, Wrc, 2 * GH2)
    num23, den2, den3 = _edge(xlc, xrc, sdw128, attc, heads=2)
    num2 = num23[:, :, :GH2]
    num3 = num23[:, :, GH2:]

    mu, logvar = _combine2(num2, den2, bi2, num3, den3, bi3)
    z = jnp.concatenate([feat_x, mu], axis=1)
    de_feat, q = _decoder(z, Wd, bd, gd, bed, cluster)
    return (z, mu, logvar, de_feat, q, feat_x, mu)


# speedup vs baseline: 1.2361x; 1.0234x over previous
import functools

import jax
import jax.numpy as jnp
from jax import lax
from jax.experimental import pallas as pl
from jax.experimental.pallas import tpu as pltpu
from jax.experimental.pallas import tpu_sc as plsc

N = 10000
E = 320000
D_IN = 128
FH1 = 256
FH2 = 128
GH1 = 128
GH2 = 64
NC = 10
LAT = GH2 + FH2

_F32 = jnp.float32


def _elu(h):
    return jnp.where(h > 0, h, jnp.exp(h) - 1.0)


def _bn(h, g, b):
    m = jnp.mean(h, axis=0)
    v = jnp.mean((h - m) ** 2, axis=0)
    return g * (h - m) * jax.lax.rsqrt(v + 1e-4) + b


def _encoder_body(x_ref, w1_ref, b1_ref, g1_ref, be1_ref,
                  w2_ref, b2_ref, g2_ref, be2_ref,
                  wl_ref, wr_ref, out_ref, xl_ref, xr_ref):
    x = x_ref[...]
    h = jnp.dot(x, w1_ref[...], preferred_element_type=_F32) + b1_ref[...]
    h = _elu(_bn(h, g1_ref[...], be1_ref[...]))
    h = jnp.dot(h, w2_ref[...], preferred_element_type=_F32) + b2_ref[...]
    feat = _elu(_bn(h, g2_ref[...], be2_ref[...]))
    out_ref[...] = feat
    xl_ref[...] = jnp.dot(feat, wl_ref[...], preferred_element_type=_F32)
    xr_ref[...] = jnp.dot(feat, wr_ref[...], preferred_element_type=_F32)


def _encoder(x, W1, b1, g1, be1, W2, b2, g2, be2, Wl1, Wr1):
    return pl.pallas_call(
        _encoder_body,
        out_shape=[
            jax.ShapeDtypeStruct((N, FH2), _F32),
            jax.ShapeDtypeStruct((N, GH1), _F32),
            jax.ShapeDtypeStruct((N, GH1), _F32),
        ],
    )(x, W1, b1, g1, be1, W2, b2, g2, be2, Wl1, Wr1)


def _combine_proj_body(num_ref, den_ref, bias_ref, wl_ref, wr_ref,
                       xl_ref, xr_ref):
    num = num_ref[0] + num_ref[1]
    den = den_ref[0] + den_ref[1]
    h = num / (den[:, None] + 1e-16) + bias_ref[...]
    h = _elu(h)
    xl_ref[...] = jnp.dot(h, wl_ref[...], preferred_element_type=_F32)
    xr_ref[...] = jnp.dot(h, wr_ref[...], preferred_element_type=_F32)


def _combine_proj(num, den, bias, Wl, Wr, gh_out):
    return pl.pallas_call(
        _combine_proj_body,
        out_shape=[
            jax.ShapeDtypeStruct((N, gh_out), _F32),
            jax.ShapeDtypeStruct((N, gh_out), _F32),
        ],
    )(num, den, bias, Wl, Wr)


def _combine2_body(num2_ref, den2_ref, bi2_ref, num3_ref, den3_ref, bi3_ref,
                   mu_ref, lv_ref):
    mu_ref[...] = (num2_ref[0] + num2_ref[1]) / (
        (den2_ref[0] + den2_ref[1])[:, None] + 1e-16) + bi2_ref[...]
    lv_ref[...] = (num3_ref[0] + num3_ref[1]) / (
        (den3_ref[0] + den3_ref[1])[:, None] + 1e-16) + bi3_ref[...]


def _combine2(num2, den2, bi2, num3, den3, bi3):
    return pl.pallas_call(
        _combine2_body,
        out_shape=[jax.ShapeDtypeStruct((N, GH2), _F32)] * 2,
    )(num2, den2, bi2, num3, den3, bi3)


def _decoder_body(z_ref, wd_ref, bd_ref, gd_ref, bed_ref, cl_ref,
                  de_ref, q_ref):
    z = z_ref[...]
    h = jnp.dot(z, wd_ref[...], preferred_element_type=_F32) + bd_ref[...]
    de_ref[...] = _elu(_bn(h, gd_ref[...], bed_ref[...]))
    cl = cl_ref[...]
    d2 = (jnp.sum(z * z, axis=1, keepdims=True)
          - 2.0 * jnp.dot(z, cl.T, preferred_element_type=_F32)
          + jnp.sum(cl * cl, axis=1)[None, :])
    q = 1.0 / (1.0 + d2)
    q_ref[...] = q / jnp.sum(q, axis=1, keepdims=True)


def _decoder(z, Wd, bd, gd, bed, cluster):
    return pl.pallas_call(
        _decoder_body,
        out_shape=[
            jax.ShapeDtypeStruct((N, D_IN), _F32),
            jax.ShapeDtypeStruct((N, NC), _F32),
        ],
    )(z, Wd, bd, gd, bed, cluster)


NP = 10240
_NW = 32
_RPT = NP // 16

_CFG = {128: (32, 316)}


def _make_sc_edge(w, heads):
    _G, _TPW = _CFG[w]
    _GB = 128 // _G
    _TB = _TPW // _GB
    hw = w // heads
    mesh = plsc.VectorSubcoreMesh(core_axis_name="c", subcore_axis_name="s")

    @functools.partial(
        pl.kernel,
        mesh=mesh,
        compiler_params=pltpu.CompilerParams(
            needs_layout_passes=False,
            use_tc_tiling_on_sc=True),
        out_type=([jax.ShapeDtypeStruct((2, NP, w), _F32)]
                  + [jax.ShapeDtypeStruct((2, NP), _F32)] * heads),
        scratch_types=(
            [pltpu.VMEM((_TB, 2, 128), jnp.int32)]
            + [pltpu.VMEM((_G, w), _F32)] * 6
            + [pltpu.VMEM((_G,), _F32)] * (2 * heads)
            + [pltpu.VMEM((_G,), jnp.int32)] * 2
            + [pltpu.VMEM((w,), _F32)]
            + [pltpu.VMEM_SHARED((NP, w), _F32)]
            + [pltpu.VMEM_SHARED((NP,), _F32)] * heads
            + [pltpu.SemaphoreType.DMA] * 4),
    )
    def edge_kernel(xl_hbm, xr_hbm, sdw_hbm, att_hbm, zrow_hbm, zden_hbm,
                    *rest):
        num_hbm = rest[0]
        den_hbms = rest[1:1 + heads]
        sc = rest[1 + heads:]
        idx_all, rl0, rl1, rr0, rr1, ob0, ob1 = sc[:7]
        PBH = [(sc[7 + 2 * h], sc[8 + 2 * h]) for h in range(heads)]
        is0, is1, att_v, num_sh = sc[7 + 2 * heads:11 + 2 * heads]
        den_shs = sc[11 + 2 * heads:11 + 3 * heads]
        sg0, sg1, ss0, ss1 = sc[11 + 3 * heads:]

        cid = lax.axis_index("c")
        sid = lax.axis_index("s")
        wid = sid * 2 + cid
        r0 = sid * _RPT
        RL, RR, OB = (rl0, rl1), (rr0, rr1), (ob0, ob1)
        IS, SG, SS = (is0, is1), (sg0, sg1), (ss0, ss1)

        pltpu.sync_copy(zrow_hbm, num_sh.at[pl.ds(r0, _RPT)])
        for h in range(heads):
            pltpu.sync_copy(zden_hbm, den_shs[h].at[pl.ds(r0, _RPT)])
        pltpu.sync_copy(att_hbm, att_v)
        pltpu.sync_copy(sdw_hbm.at[wid], idx_all)
        plsc.subcore_barrier()

        lanes = lax.iota(jnp.int32, 16)

        def gather_idx(t, which):
            tb = t // _GB
            off = (t % _GB) * _G
            return idx_all.at[tb, which, pl.ds(off, _G)]

        def fire_gather_l(b, t):
            pltpu.make_async_copy(
                xl_hbm.at[gather_idx(t, 0)], RL[b], SG[b]).start()

        def fire_gather_r(b, t):
            pltpu.make_async_copy(
                xr_hbm.at[gather_idx(t, 1)], RR[b], SG[b]).start()

        def wait_gathers(b, t):
            pltpu.make_async_copy(
                xl_hbm.at[gather_idx(t, 0)], RL[b], SG[b]).wait()
            pltpu.make_async_copy(
                xr_hbm.at[gather_idx(t, 1)], RR[b], SG[b]).wait()

        def fire_den_scatters(b):
            for h in range(heads):
                pltpu.make_async_copy(
                    PBH[h][b], den_shs[h].at[IS[b]], SS[b]).start(add=True)

        def fire_num_scatter(b):
            pltpu.make_async_copy(
                OB[b], num_sh.at[IS[b]], SS[b]).start(add=True)

        def wait_scatters(b):
            pltpu.make_async_copy(OB[b], num_sh.at[IS[b]], SS[b]).wait()
            for h in range(heads):
                pltpu.make_async_copy(
                    PBH[h][b], den_shs[h].at[IS[b]], SS[b]).wait()

        def copy_scatter_idx(b, t):
            tb = t // _GB
            off = (t % _GB) * _G
            for k in range(_G // 16):
                IS[b][pl.ds(k * 16, 16)] = idx_all[
                    tb, 1, pl.ds(off + k * 16, 16)]

        def compute_dot(b):
            rl, rr = RL[b], RR[b]
            att_regs = [att_v[pl.ds(k * 16, 16)] for k in range(w // 16)]
            kph = hw // 16

            def batch_body(bb, carry2):
                e0 = bb * 16
                pvs = [jnp.zeros((16,), _F32) for _ in range(heads)]
                for i in range(16):
                    for h in range(heads):
                        acc = None
                        for k in range(h * kph, (h + 1) * kph):
                            vl = rl[e0 + i, pl.ds(k * 16, 16)]
                            vr = rr[e0 + i, pl.ds(k * 16, 16)]
                            u = vl + vr
                            u = jnp.maximum(u, 0.2 * u)
                            t = u * att_regs[k]
                            acc = t if acc is None else acc + t
                        pvs[h] = jnp.where(lanes == i, jnp.sum(acc), pvs[h])
                for h in range(heads):
                    PBH[h][b][pl.ds(e0, 16)] = jnp.exp(pvs[h])
                return carry2

            lax.fori_loop(0, _G // 16, batch_body, 0)

        def compute_scale(b):
            rl, ob = RL[b], OB[b]
            kph = hw // 16

            def batch_body(bb, carry2):
                e0 = bb * 16
                pvs = [PBH[h][b][pl.ds(e0, 16)] for h in range(heads)]
                for i in range(16):
                    for h in range(heads):
                        p = pvs[h][i]
                        for k in range(h * kph, (h + 1) * kph):
                            ob[e0 + i, pl.ds(k * 16, 16)] = (
                                rl[e0 + i, pl.ds(k * 16, 16)] * p)
                return carry2

            lax.fori_loop(0, _G // 16, batch_body, 0)

        for b in range(2):
            fire_gather_l(b, b)
            fire_gather_r(b, b)

        def outer_body(t2, carry):
            for b in range(2):
                t = 2 * t2 + b
                wait_gathers(b, t)

                @pl.when(t2 > 0)
                def _():
                    wait_scatters(b)

                copy_scatter_idx(b, t)
                compute_dot(b)

                @pl.when(t2 < (_TPW // 2) - 1)
                def _():
                    fire_gather_r(b, t + 2)

                fire_den_scatters(b)
                compute_scale(b)

                @pl.when(t2 < (_TPW // 2) - 1)
                def _():
                    fire_gather_l(b, t + 2)

                fire_num_scatter(b)
            return carry

        lax.fori_loop(0, _TPW // 2, outer_body, 0)
        for b in range(2):
            wait_scatters(b)

        plsc.subcore_barrier()
        pltpu.sync_copy(num_sh.at[pl.ds(r0, _RPT)],
                        num_hbm.at[cid, pl.ds(r0, _RPT)])
        for h in range(heads):
            pltpu.sync_copy(den_shs[h].at[pl.ds(r0, _RPT)],
                            den_hbms[h].at[cid, pl.ds(r0, _RPT)])

    return edge_kernel


@functools.lru_cache(maxsize=None)
def _get_sc_edge(w, heads):
    return _make_sc_edge(w, heads)


def _pack_edges(src, dst, w):
    g, tpw = _CFG[w]
    gb = 128 // g
    ep = _NW * tpw * g
    pad = jnp.full((ep - E,), N, jnp.int32)
    src_p = jnp.concatenate([src, pad])
    dst_p = jnp.concatenate([dst, pad])
    sd = jnp.stack([src_p.reshape(ep // g, g), dst_p.reshape(ep // g, g)],
                   axis=1)
    sd = sd.reshape(tpw, _NW, 2, g).transpose(1, 0, 2, 3)
    return sd.reshape(_NW, tpw // gb, gb, 2, g).transpose(
        0, 1, 3, 2, 4).reshape(_NW, tpw // gb, 2, 128)


def _edge(xl, xr, sdw, att, heads=1):
    w = int(xl.shape[1])
    fn = _get_sc_edge(w, heads)
    xl_p = jnp.pad(xl, ((0, NP - N), (0, 0)))
    xr_p = jnp.pad(xr, ((0, NP - N), (0, 0)))
    zrow = jnp.zeros((_RPT, w), _F32)
    zden = jnp.zeros((_RPT,), _F32)
    outs = fn(xl_p, xr_p, sdw, att, zrow, zden)
    num = outs[0][:, :N]
    dens = [d[:, :N] for d in outs[1:]]
    return (num, *dens)


def kernel(x, adj, x_hr, training, W1, b1, g1, be1, W2, b2, g2, be2,
           Wl1, Wr1, a1, bi1, Wlh, Wrh, ah, bih,
           Wl2, Wr2, a2, bi2, Wl3, Wr3, a3, bi3,
           Wd, bd, gd, bed, cluster):
    sdw128 = _pack_edges(adj[0], adj[1], GH1)

    feat_x, xl1, xr1 = _encoder(x, W1, b1, g1, be1, W2, b2, g2, be2, Wl1, Wr1)

    num1, den1 = _edge(xl1, xr1, sdw128, a1)
    xlh, xrh = _combine_proj(num1, den1, bi1, Wlh, Wrh, GH1)

    numh, denh = _edge(xlh, xrh, sdw128, ah)
    Wlc = jnp.concatenate([Wl2, Wl3], axis=1)
    Wrc = jnp.concatenate([Wr2, Wr3], axis=1)
    attc = jnp.concatenate([a2, a3])
    xlc, xrc = _combine_proj(numh, denh, bih, Wlc, Wrc, 2 * GH2)
    num23, den2, den3 = _edge(xlc, xrc, sdw128, attc, heads=2)
    num2 = num23[:, :, :GH2]
    num3 = num23[:, :, GH2:]

    mu, logvar = _combine2(num2, den2, bi2, num3, den3, bi3)
    z = jnp.concatenate([feat_x, mu], axis=1)
    de_feat, q = _decoder(z, Wd, bd, gd, bed, cluster)
    return (z, mu, logvar, de_feat, q, feat_x, mu)
